# baseline (speedup 1.0000x reference)
import jax
import jax.numpy as jnp
from jax import lax
from jax.experimental import pallas as pl
from jax.experimental.pallas import tpu as pltpu
from jax.experimental.pallas import tpu_sc as plsc

_B, _N, _D = 16, 196, 768
_H, _E, _K, _DM = 16, 64, 2048, 64
_BN = _B * _N
_BNP = 3328
_T = 256
_RB = 64
_KT = 128
_PREC = jax.lax.Precision.DEFAULT

_NW = 32
_TPW = _BNP // _NW
_R0, _R1 = 56, 48


def _tc_block(emb_ref, rp_ref, cb_ref, *refs):
    idx_refs = refs[:_H]
    pcat_ref, cbt_ref, esq_ref = refs[_H:]
    @pl.when(pl.program_id(0) == 0)
    def _():
        pcat_ref[...] = 2.0 * jnp.concatenate(
            [rp_ref[h] for h in range(_H)], axis=1)
        for h in range(_H):
            cbt_ref[h] = cb_ref[h].T
        esq_ref[...] = jnp.sum(cbt_ref[...] ** 2, axis=1)

    x2 = jnp.dot(emb_ref[...], pcat_ref[...], precision=_PREC)
    lane = lax.broadcasted_iota(jnp.int32, (_RB, _KT), 1)
    for h in range(_H):
        dots2 = jnp.dot(x2[:, h * _E:(h + 1) * _E], cbt_ref[h],
                        precision=_PREC)
        for rb in range(_T // _RB):
            sub = dots2[rb * _RB:(rb + 1) * _RB]
            best = sub[:, 0:_KT] - esq_ref[h][None, 0:_KT]
            bestkt = jnp.zeros((_RB, _KT), jnp.int32)
            for kt in range(1, _K // _KT):
                tile = (sub[:, kt * _KT:(kt + 1) * _KT]
                        - esq_ref[h][None, kt * _KT:(kt + 1) * _KT])
                upd = tile > best
                best = jnp.where(upd, tile, best)
                bestkt = jnp.where(upd, kt, bestkt)
            mx = jnp.max(best, axis=1, keepdims=True)
            cand = jnp.where(best == mx, bestkt * _KT + lane, _K)
            piece = jnp.min(cand, axis=1) + h * _K
            idx_refs[h][rb * _RB:(rb + 1) * _RB] = piece


def _sc_gather_mean(vals_hbm, *refs):
    idx_hbms = refs[:_H]
    out_hbm = refs[_H]
    idx_v, rows_v, out_v, sem = refs[_H + 1:]
    wid = lax.axis_index("s") * 2 + lax.axis_index("c")
    base = wid * _TPW

    for r, (off, tpr) in enumerate(((0, _R0), (_R0, _R1))):
        for h in range(_H):
            pltpu.sync_copy(idx_hbms[h].at[pl.ds(base + off, tpr)],
                            idx_v.at[pl.ds(h * tpr, tpr)])
        pltpu.async_copy(
            vals_hbm.at[idx_v.at[pl.ds(0, tpr * _H)]],
            rows_v.at[pl.ds(0, tpr * _H)], sem).wait()

        @pl.loop(0, tpr)
        def _(t):
            for c in range(0, _DM, 16):
                acc = rows_v.at[t, pl.ds(c, 16)][...]
                for h in range(1, _H):
                    acc = acc + rows_v.at[h * tpr + t, pl.ds(c, 16)][...]
                out_v.at[pl.ds((off + t) * _DM + c, 16)][...] = (
                    acc * (1.0 / _H))

    pltpu.sync_copy(out_v, out_hbm.at[pl.ds(base * _DM, _TPW * _DM)])


def kernel(embeddings, rand_proj, codebook, values):
    emb = jnp.pad(embeddings.reshape(_BN, _D), ((0, _BNP - _BN), (0, 0)))

    idx_list = pl.pallas_call(
        _tc_block,
        grid=(_BNP // _T,),
        in_specs=[
            pl.BlockSpec((_T, _D), lambda i: (i, 0)),
            pl.BlockSpec((_H, _D, _E), lambda i: (0, 0, 0)),
            pl.BlockSpec((_H, _K, _E), lambda i: (0, 0, 0)),
        ],
        out_specs=[pl.BlockSpec((_T,), lambda i: (i,)) for _ in range(_H)],
        out_shape=[jax.ShapeDtypeStruct((_BNP,), jnp.int32)
                   for _ in range(_H)],
        scratch_shapes=[
            pltpu.VMEM((_D, _H * _E), jnp.float32),
            pltpu.VMEM((_H, _E, _K), jnp.float32),
            pltpu.VMEM((_H, _K), jnp.float32),
        ],
        compiler_params=pltpu.CompilerParams(
            dimension_semantics=("arbitrary",)),
    )(emb, rand_proj, codebook)

    vals_flat = jnp.pad(values.reshape(_H * _K, _DM), ((0, 0), (0, 128 - _DM)))
    mesh = plsc.VectorSubcoreMesh(core_axis_name="c", subcore_axis_name="s")
    sc = pl.kernel(
        _sc_gather_mean,
        mesh=mesh,
        out_type=jax.ShapeDtypeStruct((_BNP * _DM,), jnp.float32),
        scratch_types=[
            pltpu.VMEM((_R0 * _H,), jnp.int32),
            pltpu.VMEM((_R0 * _H, 128), jnp.float32),
            pltpu.VMEM((_TPW * _DM,), jnp.float32),
            pltpu.SemaphoreType.DMA,
        ],
    )
    out = sc(vals_flat, *idx_list)
    return out.reshape(_BNP, _DM)[:_BN].reshape(_B, _N, _DM)
